# Initial kernel scaffold; baseline (speedup 1.0000x reference)
import jax
import jax.numpy as jnp
from jax.experimental import pallas as pl


def kernel(feat, edge_index, edge_weight, W, prelu_w):
    raise NotImplementedError("write your pallas kernel here")


# speedup vs baseline: 3.6041x; 3.6041x over previous
import functools

import jax
import jax.numpy as jnp
from jax import lax
from jax.experimental import pallas as pl
from jax.experimental.pallas import tpu as pltpu
from jax.experimental.pallas import tpu_sc as plsc

N_NODES = 10000
FEAT = 128
N_EDGES = 320000

NC = 2
NS = 16
LANES = 16

CHUNK = 128
K_CHUNKS = 79
EDGES_PER_WORKER = CHUNK * K_CHUNKS
E_PAD = EDGES_PER_WORKER * NC * NS
ACC_N = 10240
ROWS_PER_SUBCORE = ACC_N // NS
ZB_ROWS = 128


def _matmul_body(f_ref, wt_ref, o_ref):
    o_ref[...] = jnp.dot(f_ref[...], wt_ref[...],
                         preferred_element_type=jnp.float32)


def _matmul(feat, Wt):
    blk = 1000
    return pl.pallas_call(
        _matmul_body,
        grid=(N_NODES // blk,),
        in_specs=[
            pl.BlockSpec((blk, FEAT), lambda i: (i, 0)),
            pl.BlockSpec((FEAT, FEAT), lambda i: (0, 0)),
        ],
        out_specs=pl.BlockSpec((blk, FEAT), lambda i: (i, 0)),
        out_shape=jax.ShapeDtypeStruct((N_NODES, FEAT), jnp.float32),
    )(feat, Wt)


def _edge_body(h_hbm, row_hbm, col_hbm, ew_hbm, out_hbm,
               rowv, colv, ewv, msgv, zbv, acc, sem):
    core = lax.axis_index("c")
    sid = lax.axis_index("s")
    wid = core * NS + sid

    @pl.loop(0, ZB_ROWS)
    def _(i):
        @pl.loop(0, FEAT, step=LANES)
        def _(j):
            zbv[i, pl.ds(j, LANES)] = jnp.zeros((LANES,), jnp.float32)

    @pl.loop(0, ROWS_PER_SUBCORE, step=ZB_ROWS)
    def _(r):
        pltpu.sync_copy(zbv, acc.at[pl.ds(sid * ROWS_PER_SUBCORE + r, ZB_ROWS)])

    plsc.subcore_barrier()

    base = wid * EDGES_PER_WORKER

    @pl.loop(0, K_CHUNKS)
    def _(c):
        off = base + c * CHUNK
        pltpu.sync_copy(row_hbm.at[pl.ds(off, CHUNK)], rowv)
        pltpu.sync_copy(col_hbm.at[pl.ds(off, CHUNK)], colv)
        pltpu.sync_copy(ew_hbm.at[pl.ds(off, CHUNK)], ewv)
        pltpu.async_copy(h_hbm.at[rowv], msgv, sem).wait()

        @pl.loop(0, CHUNK, step=LANES)
        def _(e0):
            w16 = ewv[pl.ds(e0, LANES)]
            for l in range(LANES):
                wvec = jnp.full((LANES,), w16[l], jnp.float32)
                for j in range(FEAT // LANES):
                    sl = pl.ds(j * LANES, LANES)
                    msgv[e0 + l, sl] = msgv[e0 + l, sl] * wvec

        pltpu.sync_copy(msgv, acc.at[colv], add=True)

    plsc.subcore_barrier()

    @pl.loop(0, ROWS_PER_SUBCORE, step=ZB_ROWS)
    def _(r):
        rr = sid * ROWS_PER_SUBCORE + r
        pltpu.sync_copy(acc.at[pl.ds(rr, ZB_ROWS)],
                        out_hbm.at[core, pl.ds(rr, ZB_ROWS)])


def _edge_scatter(h, row, col, ew):
    mesh = plsc.VectorSubcoreMesh(core_axis_name="c", subcore_axis_name="s")
    kern = pl.kernel(
        _edge_body,
        out_type=jax.ShapeDtypeStruct((NC, ACC_N, FEAT), jnp.float32),
        mesh=mesh,
        scratch_types=[
            pltpu.VMEM((CHUNK,), jnp.int32),
            pltpu.VMEM((CHUNK,), jnp.int32),
            pltpu.VMEM((CHUNK,), jnp.float32),
            pltpu.VMEM((CHUNK, FEAT), jnp.float32),
            pltpu.VMEM((ZB_ROWS, FEAT), jnp.float32),
            pltpu.VMEM_SHARED((ACC_N, FEAT), jnp.float32),
            pltpu.SemaphoreType.DMA,
        ],
    )
    return kern(h, row, col, ew)


def _combine_body(p_ref, a_ref, o_ref):
    s = p_ref[0] + p_ref[1]
    o_ref[...] = jnp.where(s >= 0, s, a_ref[0] * s)


def _combine(partial, prelu_w):
    blk = 1000
    return pl.pallas_call(
        _combine_body,
        grid=(N_NODES // blk,),
        in_specs=[
            pl.BlockSpec((NC, blk, FEAT), lambda i: (0, i, 0)),
            pl.BlockSpec(memory_space=pltpu.SMEM),
        ],
        out_specs=pl.BlockSpec((blk, FEAT), lambda i: (i, 0)),
        out_shape=jax.ShapeDtypeStruct((N_NODES, FEAT), jnp.float32),
    )(partial, prelu_w.reshape(1))


def kernel(feat, edge_index, edge_weight, W, prelu_w):
    row = edge_index[0].astype(jnp.int32)
    col = edge_index[1].astype(jnp.int32)
    pad = E_PAD - N_EDGES
    row = jnp.pad(row, (0, pad))
    col = jnp.pad(col, (0, pad))
    ew = jnp.pad(edge_weight.astype(jnp.float32), (0, pad))

    h = _matmul(feat, W.T)
    partial = _edge_scatter(h, row, col, ew)
    return _combine(partial, prelu_w)
